# baseline (speedup 1.0000x reference)
import functools

import jax
import jax.numpy as jnp
from jax import lax
from jax.experimental import pallas as pl
from jax.experimental.pallas import tpu as pltpu
from jax.experimental.pallas import tpu_sc as plsc

DIM_ = 32
NLIM = DIM_ - 1


def _binidx_body(xa_ref, xb_ref, idx_ref):
    xa = xa_ref[0]
    xb = xb_ref[0]
    s = None
    for c in range(3):
        d = xa[:, c : c + 1] - xb[c : c + 1, :]
        t = d * d
        s = t if s is None else s + t
    dist = jnp.sqrt(s)
    idx = jnp.clip(jnp.ceil(dist * 1.5), 0.0, 31.0).astype(jnp.int32)
    idx_ref[0] = idx


def _bin_indices(x_t):
    b, n, _ = x_t.shape
    pad = jnp.zeros((b, n, 5), x_t.dtype)
    xa = jnp.concatenate([x_t, pad], axis=-1)
    xb = jnp.transpose(xa, (0, 2, 1))
    return pl.pallas_call(
        _binidx_body,
        grid=(b,),
        in_specs=[
            pl.BlockSpec((1, n, 8), lambda i: (i, 0, 0)),
            pl.BlockSpec((1, 8, n), lambda i: (i, 0, 0)),
        ],
        out_specs=pl.BlockSpec((1, n, n), lambda i: (i, 0, 0)),
        out_shape=jax.ShapeDtypeStruct((b, n, n), jnp.int32),
    )(xa, xb)


NC = 2
NS = 16
NW = NC * NS
CH = 1024
LANES = 16


def _make_expand(btot, n):
    nface = btot // n
    fpc = CH // n
    b_per_w = btot // NW
    nchunk = b_per_w // CH
    f_per_w = nface // NW
    mesh = plsc.VectorSubcoreMesh(
        core_axis_name="c", subcore_axis_name="s", num_cores=NC, num_subcores=NS
    )

    @functools.partial(
        pl.kernel,
        out_type=jax.ShapeDtypeStruct((nface, DIM_, n), jnp.float32),
        mesh=mesh,
        compiler_params=pltpu.CompilerParams(
            use_tc_tiling_on_sc=True, needs_layout_passes=False
        ),
        scratch_types=[
            pltpu.VMEM((f_per_w, n), jnp.int32),
            pltpu.VMEM((fpc, DIM_, n), jnp.float32),
            pltpu.VMEM((fpc, DIM_, n), jnp.float32),
            pltpu.SemaphoreType.DMA,
            pltpu.SemaphoreType.DMA,
        ],
    )
    def expand(idx_hbm, out_hbm, idx_v, rows_a, rows_b, wsem, isem):
        wid = lax.axis_index("s") * NC + lax.axis_index("c")
        idx_cp = pltpu.async_copy(idx_hbm.at[wid], idx_v, isem)

        zeros16 = jnp.zeros((LANES,), jnp.float32)
        ones16 = jnp.ones((LANES,), jnp.float32)
        lane = lax.iota(jnp.int32, LANES)
        gpf = n // LANES

        bufs = (rows_a, rows_b)
        for buf in bufs:
            def zbody(i, c, buf=buf):
                f = i // (DIM_ * gpf)
                r = (i // gpf) % DIM_
                col = (i % gpf) * LANES
                buf[f, r, pl.ds(col, LANES)] = zeros16
                return c

            lax.fori_loop(0, fpc * DIM_ * gpf, zbody, 0)
        idx_cp.wait()

        fbase = pl.multiple_of(wid * f_per_w, f_per_w)
        for k in range(nchunk):
            buf = bufs[k % 2]
            if k >= 2:
                pltpu.make_async_copy(
                    buf, out_hbm.at[pl.ds(fbase, fpc)], wsem
                ).wait()
            km2 = max(k - 2, 0)

            def cbody(g, c, buf=buf, k=k, km2=km2):
                frel = g // gpf
                col = (g % gpf) * LANES
                old = idx_v[km2 * fpc + frel, pl.ds(col, LANES)]
                new = idx_v[k * fpc + frel, pl.ds(col, LANES)]
                fvec = jnp.full((LANES,), frel, jnp.int32)
                jvec = lane + col
                plsc.store_scatter(buf, [fvec, old, jvec], zeros16)
                plsc.store_scatter(buf, [fvec, new, jvec], ones16)
                return c

            lax.fori_loop(0, CH // LANES, cbody, 0)
            pltpu.async_copy(
                buf,
                out_hbm.at[pl.ds(pl.multiple_of(fbase + k * fpc, fpc), fpc)],
                wsem,
            )
        for buf in bufs:
            pltpu.make_async_copy(
                buf, out_hbm.at[pl.ds(fbase, fpc)], wsem
            ).wait()

    return expand


def kernel(x_t):
    b, n, _ = x_t.shape
    idx = _bin_indices(x_t)
    btot = b * n * n
    idx3 = idx.reshape(NW, btot // (NW * n), n)
    out = _make_expand(btot, n)(idx3)
    return jnp.transpose(out.reshape(b, n, DIM_, n), (0, 1, 3, 2))


# speedup vs baseline: 1.0160x; 1.0160x over previous
import functools

import jax
import jax.numpy as jnp
from jax import lax
from jax.experimental import pallas as pl
from jax.experimental.pallas import tpu as pltpu
from jax.experimental.pallas import tpu_sc as plsc

DIM_ = 32
NLIM = DIM_ - 1


def _binidx_body(xa_ref, idx_ref):
    xa = xa_ref[0]
    xb = jnp.transpose(xa, (1, 0))
    s = None
    for c in range(3):
        d = xa[:, c : c + 1] - xb[c : c + 1, :]
        t = d * d
        s = t if s is None else s + t
    dist = jnp.sqrt(s)
    idx = jnp.clip(jnp.ceil(dist * 1.5), 0.0, 31.0).astype(jnp.int32)
    idx_ref[0] = idx


def _bin_indices(x_t):
    b, n, _ = x_t.shape
    pad = jnp.zeros((b, n, 5), x_t.dtype)
    xa = jnp.concatenate([x_t, pad], axis=-1)
    return pl.pallas_call(
        _binidx_body,
        grid=(b,),
        in_specs=[
            pl.BlockSpec((1, n, 8), lambda i: (i, 0, 0)),
        ],
        out_specs=pl.BlockSpec((1, n, n), lambda i: (i, 0, 0)),
        out_shape=jax.ShapeDtypeStruct((b, n, n), jnp.int32),
    )(xa)


NC = 2
NS = 16
NW = NC * NS
CH = 1024
LANES = 16


def _make_expand(btot, n):
    nface = btot // n
    fpc = CH // n
    b_per_w = btot // NW
    nchunk = b_per_w // CH
    f_per_w = nface // NW
    mesh = plsc.VectorSubcoreMesh(
        core_axis_name="c", subcore_axis_name="s", num_cores=NC, num_subcores=NS
    )

    @functools.partial(
        pl.kernel,
        out_type=jax.ShapeDtypeStruct((nface, DIM_, n), jnp.float32),
        mesh=mesh,
        compiler_params=pltpu.CompilerParams(
            use_tc_tiling_on_sc=True, needs_layout_passes=False
        ),
        scratch_types=[
            pltpu.VMEM((f_per_w, n), jnp.int32),
            pltpu.VMEM((fpc, DIM_, n), jnp.float32),
            pltpu.VMEM((fpc, DIM_, n), jnp.float32),
            pltpu.SemaphoreType.DMA,
            pltpu.SemaphoreType.DMA,
        ],
    )
    def expand(idx_hbm, out_hbm, idx_v, rows_a, rows_b, wsem, isem):
        wid = lax.axis_index("s") * NC + lax.axis_index("c")
        idx_cp = pltpu.async_copy(idx_hbm.at[wid], idx_v, isem)

        zeros16 = jnp.zeros((LANES,), jnp.float32)
        ones16 = jnp.ones((LANES,), jnp.float32)
        lane = lax.iota(jnp.int32, LANES)
        gpf = n // LANES

        bufs = (rows_a, rows_b)

        def zero_window(buf):
            def zbody(i, c, buf=buf):
                f = i // (DIM_ * gpf)
                r = (i // gpf) % DIM_
                col = (i % gpf) * LANES
                buf[f, r, pl.ds(col, LANES)] = zeros16
                return c

            lax.fori_loop(0, fpc * DIM_ * gpf, zbody, 0)

        zero_window(rows_a)
        idx_cp.wait()

        fbase = pl.multiple_of(wid * f_per_w, f_per_w)
        for k in range(nchunk):
            buf = bufs[k % 2]
            if k >= 2:
                pltpu.make_async_copy(
                    buf, out_hbm.at[pl.ds(fbase, fpc)], wsem
                ).wait()
            km2 = max(k - 2, 0)

            def cbody(g, c, buf=buf, k=k, km2=km2):
                frel = g // gpf
                col = (g % gpf) * LANES
                old = idx_v[km2 * fpc + frel, pl.ds(col, LANES)]
                new = idx_v[k * fpc + frel, pl.ds(col, LANES)]
                fvec = jnp.full((LANES,), frel, jnp.int32)
                jvec = lane + col
                plsc.store_scatter(buf, [fvec, old, jvec], zeros16)
                plsc.store_scatter(buf, [fvec, new, jvec], ones16)
                return c

            lax.fori_loop(0, CH // LANES, cbody, 0)
            pltpu.async_copy(
                buf,
                out_hbm.at[pl.ds(pl.multiple_of(fbase + k * fpc, fpc), fpc)],
                wsem,
            )
            if k == 0:
                zero_window(rows_b)
        for buf in bufs:
            pltpu.make_async_copy(
                buf, out_hbm.at[pl.ds(fbase, fpc)], wsem
            ).wait()

    return expand


def kernel(x_t):
    b, n, _ = x_t.shape
    idx = _bin_indices(x_t)
    btot = b * n * n
    idx3 = idx.reshape(NW, btot // (NW * n), n)
    out = _make_expand(btot, n)(idx3)
    return jnp.transpose(out.reshape(b, n, DIM_, n), (0, 1, 3, 2))


# speedup vs baseline: 1.0162x; 1.0002x over previous
import functools

import jax
import jax.numpy as jnp
from jax import lax
from jax.experimental import pallas as pl
from jax.experimental.pallas import tpu as pltpu
from jax.experimental.pallas import tpu_sc as plsc

DIM_ = 32


def _binidx_body(xa_ref, idx_ref):
    xa = xa_ref[0]
    xb = jnp.transpose(xa, (1, 0))
    s = None
    for c in range(3):
        d = xa[:, c : c + 1] - xb[c : c + 1, :]
        t = d * d
        s = t if s is None else s + t
    dist = jnp.sqrt(s)
    idx = jnp.clip(jnp.ceil(dist * 1.5), 0.0, 31.0).astype(jnp.int32)
    idx_ref[0] = idx


def _bin_indices(x_t):
    b, n, _ = x_t.shape
    pad = jnp.zeros((b, n, 5), x_t.dtype)
    xa = jnp.concatenate([x_t, pad], axis=-1)
    return pl.pallas_call(
        _binidx_body,
        grid=(b,),
        in_specs=[
            pl.BlockSpec((1, n, 8), lambda i: (i, 0, 0)),
        ],
        out_specs=pl.BlockSpec((1, n, n), lambda i: (i, 0, 0)),
        out_shape=jax.ShapeDtypeStruct((b, n, n), jnp.int32),
    )(xa)


NC = 2
NS = 16
NW = NC * NS
CH = 1024
LANES = 16


def _make_expand(btot, n):
    nface = btot // n
    fpc = CH // n
    b_per_w = btot // NW
    nchunk = b_per_w // CH
    f_per_w = nface // NW
    mesh = plsc.VectorSubcoreMesh(
        core_axis_name="c", subcore_axis_name="s", num_cores=NC, num_subcores=NS
    )

    @functools.partial(
        pl.kernel,
        out_type=jax.ShapeDtypeStruct((nface, DIM_, n), jnp.float32),
        mesh=mesh,
        compiler_params=pltpu.CompilerParams(
            use_tc_tiling_on_sc=True, needs_layout_passes=False
        ),
        scratch_types=[
            pltpu.VMEM((f_per_w, n), jnp.int32),
            pltpu.VMEM((fpc, DIM_, n), jnp.float32),
            pltpu.VMEM((fpc, DIM_, n), jnp.float32),
            pltpu.SemaphoreType.DMA,
            pltpu.SemaphoreType.DMA,
        ],
    )
    def expand(idx_hbm, out_hbm, idx_v, rows_a, rows_b, wsem, isem):
        wid = lax.axis_index("s") * NC + lax.axis_index("c")
        idx_cp = pltpu.async_copy(idx_hbm.at[wid], idx_v, isem)

        zeros16 = jnp.zeros((LANES,), jnp.float32)
        ones16 = jnp.ones((LANES,), jnp.float32)
        lane = lax.iota(jnp.int32, LANES)
        gpf = n // LANES

        bufs = (rows_a, rows_b)

        def zero_window(buf):
            def zbody(i, c, buf=buf):
                f = i // (DIM_ * gpf)
                r = (i // gpf) % DIM_
                col = (i % gpf) * LANES
                buf[f, r, pl.ds(col, LANES)] = zeros16
                return c

            lax.fori_loop(0, fpc * DIM_ * gpf, zbody, 0)

        zero_window(rows_a)
        idx_cp.wait()

        fbase = pl.multiple_of(wid * f_per_w, f_per_w)
        for k in range(nchunk):
            buf = bufs[k % 2]
            if k >= 2:
                pltpu.make_async_copy(
                    buf, out_hbm.at[pl.ds(fbase, fpc)], wsem
                ).wait()
            km2 = max(k - 2, 0)

            def cbody(g, c, buf=buf, k=k, km2=km2):
                frel = g // gpf
                col = (g % gpf) * LANES
                old = idx_v[km2 * fpc + frel, pl.ds(col, LANES)]
                new = idx_v[k * fpc + frel, pl.ds(col, LANES)]
                fvec = jnp.full((LANES,), frel, jnp.int32)
                jvec = lane + col
                plsc.store_scatter(buf, [fvec, old, jvec], zeros16)
                plsc.store_scatter(buf, [fvec, new, jvec], ones16)
                return c

            lax.fori_loop(0, CH // LANES, cbody, 0)
            pltpu.async_copy(
                buf,
                out_hbm.at[pl.ds(pl.multiple_of(fbase + k * fpc, fpc), fpc)],
                wsem,
            )
            if k == 0:
                zero_window(rows_b)
        for buf in bufs:
            pltpu.make_async_copy(
                buf, out_hbm.at[pl.ds(fbase, fpc)], wsem
            ).wait()

    return expand


def kernel(x_t):
    b, n, _ = x_t.shape
    idx = _bin_indices(x_t)
    btot = b * n * n
    idx3 = idx.reshape(NW, btot // (NW * n), n)
    out = _make_expand(btot, n)(idx3)
    return jnp.transpose(out.reshape(b, n, DIM_, n), (0, 1, 3, 2))
